# Initial kernel scaffold; baseline (speedup 1.0000x reference)
import jax
import jax.numpy as jnp
from jax.experimental import pallas as pl


def kernel(x):
    raise NotImplementedError("write your pallas kernel here")


# speedup vs baseline: 3.2204x; 3.2204x over previous
import jax
import jax.numpy as jnp
from jax.experimental import pallas as pl


_ROWS, _COLS = 128, 32768
_BLOCK_ROWS = 8


def _onehot_argmax_kernel(x_ref, o_ref):
    xb = x_ref[...]
    m = jnp.max(xb, axis=-1, keepdims=True)
    iota = jax.lax.broadcasted_iota(jnp.int32, xb.shape, 1)
    big = jnp.int32(2**30)
    first = jnp.min(jnp.where(xb == m, iota, big), axis=-1, keepdims=True)
    o_ref[...] = (iota == first).astype(jnp.float32)


def kernel(x):
    grid = (_ROWS // _BLOCK_ROWS,)
    return pl.pallas_call(
        _onehot_argmax_kernel,
        out_shape=jax.ShapeDtypeStruct((_ROWS, _COLS), jnp.float32),
        grid=grid,
        in_specs=[pl.BlockSpec((_BLOCK_ROWS, _COLS), lambda i: (i, 0))],
        out_specs=pl.BlockSpec((_BLOCK_ROWS, _COLS), lambda i: (i, 0)),
    )(x)
